# baseline (speedup 1.0000x reference)
import functools

import jax
import jax.numpy as jnp
from jax import lax
from jax.experimental import pallas as pl
from jax.experimental.pallas import tpu as pltpu
from jax.experimental.pallas import tpu_sc as plsc

_B = 131072
_NW = 32
_ROWS_PER_W = _B // _NW
_CHUNK = 256
_NPASS = _CHUNK // 16
_NCHUNK = _ROWS_PER_W // _CHUNK


_CW = _CHUNK * 128
_OW = _CHUNK * 64


def _sc_body(x_hbm, out_hbm, xv, ov, ocm, hist, mv, bv, isem, osem):
    nc = 2
    wid = lax.axis_index("s") * nc + lax.axis_index("c")
    iota = lax.iota(jnp.int32, 16)
    sixteens_i = jnp.full((16,), 16, jnp.int32)
    zeros_i = jnp.zeros((16,), jnp.int32)

    for p in range(65):
        hist[pl.ds(p * 16, 16)] = zeros_i

    pltpu.async_copy(
        x_hbm.at[pl.ds(wid * _ROWS_PER_W * 128, _CW)], xv.at[pl.ds(0, _CW)],
        isem)

    def chunk_body(ci):
        base = wid * _ROWS_PER_W + ci * _CHUNK
        ibuf = (ci & 1) * _CW
        obuf = (ci & 1) * _OW
        pltpu.make_async_copy(
            x_hbm.at[pl.ds(0, _CW)], xv.at[pl.ds(0, _CW)], isem).wait()

        @pl.when(ci + 1 < _NCHUNK)
        def _prefetch():
            pltpu.async_copy(
                x_hbm.at[pl.ds((base + _CHUNK) * 128, _CW)],
                xv.at[pl.ds(_CW - ibuf, _CW)],
                isem)

        @pl.when(ci >= 2)
        def _drain_out():
            pltpu.make_async_copy(
                out_hbm.at[pl.ds(0, _OW // 128)],
                ov.at[pl.ds(0, _OW // 128)], osem).wait()

        def pass_body(pi):
            rowf = ibuf + (iota + pi * 16) * 128
            rowo = obuf + (iota + pi * 16) * 64

            @plsc.parallel_loop(
                0, 64,
                unroll=16,
                carry=(jnp.zeros((16,), jnp.float32),
                       jnp.zeros((16,), jnp.float32)),
            )
            def kloop(k, c):
                x_lo, y_lo = c
                col = (iota + k) & 63
                a = rowf + col
                x_hi = plsc.load_gather(xv, [a])
                y_hi = plsc.load_gather(xv, [a + 64])
                s = x_hi * 64.0
                ti = s.astype(jnp.int32)
                u = ti + (ti.astype(jnp.float32) < s).astype(jnp.int32)
                plsc.addupdate_scatter(hist, [u * 16 + iota], sixteens_i)
                m = (y_hi - y_lo) / (x_hi - x_lo)
                b = y_lo - m * x_lo
                maddr = col * 16 + iota
                plsc.store_scatter(mv, [maddr], m)
                plsc.store_scatter(bv, [maddr], b)
                return (x_hi, y_hi)

            x_last, y_last = kloop
            xf = plsc.load_gather(xv, [rowf + iota])
            yf = plsc.load_gather(xv, [rowf + iota + 64])
            mw = (yf - y_last) / (xf - x_last)
            bw = y_last - mw * x_last
            msk = iota > 0
            plsc.store_scatter(mv, [iota * 17], mw, mask=msk)
            plsc.store_scatter(bv, [iota * 17], bw, mask=msk)
            x0 = plsc.load_gather(xv, [rowf + 0])
            y0 = plsc.load_gather(xv, [rowf + 64])
            m0 = (y0 - 0.0) / (x0 - (-0.01))
            mv[pl.ds(0, 16)] = m0
            bv[pl.ds(0, 16)] = 0.0 - m0 * (-0.01)
            x63 = plsc.load_gather(xv, [rowf + 63])
            y63 = plsc.load_gather(xv, [rowf + 127])
            m64 = (0.0 - y63) / (1.01 - x63)
            mv[pl.ds(64 * 16, 16)] = m64
            bv[pl.ds(64 * 16, 16)] = y63 - m64 * x63

            @plsc.parallel_loop(0, 64, unroll=8, carry=zeros_i)
            def ploop(p, cnt16):
                h = hist[pl.ds(p * 16, 16)]
                hist[pl.ds(p * 16, 16)] = zeros_i
                cnt16 = cnt16 + h
                a = cnt16 + iota
                mg = plsc.load_gather(mv, [a])
                bg = plsc.load_gather(bv, [a])
                t = p.astype(jnp.float32) * (1.0 / 64.0)
                o = mg * t + bg
                ocm[pl.ds(p * 16, 16)] = o
                return cnt16

            del ploop
            hist[pl.ds(64 * 16, 16)] = zeros_i

            @plsc.parallel_loop(0, 16, unroll=4)
            def tloop(d):
                e = (iota + d) & 15
                e16 = e * 16
                for blk in range(4):
                    src = e16 + iota + blk * 256
                    v = plsc.load_gather(ocm, [src])
                    dst = rowo + e + blk * 16
                    plsc.store_scatter(ov, [dst >> 7, dst & 127], v)

            del tloop

        pl.loop(0, _NPASS)(pass_body)
        pltpu.async_copy(
            ov.at[pl.ds(pl.multiple_of(obuf // 128, 128), _OW // 128)],
            out_hbm.at[pl.ds(pl.multiple_of(base // 2, 128), _OW // 128)],
            osem)

    pl.loop(0, _NCHUNK)(chunk_body)
    for _ in range(2):
        pltpu.make_async_copy(
            out_hbm.at[pl.ds(0, _OW // 128)],
            ov.at[pl.ds(0, _OW // 128)], osem).wait()


def kernel(X):
    mesh = plsc.VectorSubcoreMesh(core_axis_name="c", subcore_axis_name="s")
    f = functools.partial(
        pl.kernel,
        mesh=mesh,
        compiler_params=pltpu.CompilerParams(needs_layout_passes=False),
        out_type=jax.ShapeDtypeStruct((_B // 2, 128), jnp.float32),
        scratch_types=[
            pltpu.VMEM((2 * _CW,), jnp.float32),
            pltpu.VMEM((2 * _OW // 128, 128), jnp.float32),
            pltpu.VMEM((64 * 16,), jnp.float32),
            pltpu.VMEM((65 * 16,), jnp.int32),
            pltpu.VMEM((65 * 16,), jnp.float32),
            pltpu.VMEM((65 * 16,), jnp.float32),
            pltpu.SemaphoreType.DMA,
            pltpu.SemaphoreType.DMA,
        ],
    )(_sc_body)
    out = f(X.reshape(_B * 128))
    return out.reshape(_B, 64)


# speedup vs baseline: 1.2892x; 1.2892x over previous
import functools

import jax
import jax.numpy as jnp
from jax import lax
from jax.experimental import pallas as pl
from jax.experimental.pallas import tpu as pltpu
from jax.experimental.pallas import tpu_sc as plsc

_B = 131072
_NW = 32
_ROWS_PER_W = _B // _NW
_CHUNK = 128
_NPASS = _CHUNK // 16
_NCHUNK = _ROWS_PER_W // _CHUNK


_CW = _CHUNK * 128
_OW = _CHUNK * 128


def _sc_body(x_hbm, out_hbm, xv, ov, ocm, hist, mv, bv, isem, osem):
    nc = 2
    wid = lax.axis_index("s") * nc + lax.axis_index("c")
    iota = lax.iota(jnp.int32, 16)
    sixteens_i = jnp.full((16,), 16, jnp.int32)
    zeros_i = jnp.zeros((16,), jnp.int32)

    for p in range(65):
        hist[pl.ds(p * 16, 16)] = zeros_i

    pltpu.async_copy(
        x_hbm.at[pl.ds(wid * _ROWS_PER_W * 128, _CW)], xv.at[pl.ds(0, _CW)],
        isem)

    def chunk_body(ci):
        base = wid * _ROWS_PER_W + ci * _CHUNK
        ibuf = (ci & 1) * _CW
        obuf = (ci & 1) * _OW
        pltpu.make_async_copy(
            x_hbm.at[pl.ds(0, _CW)], xv.at[pl.ds(0, _CW)], isem).wait()

        @pl.when(ci + 1 < _NCHUNK)
        def _prefetch():
            pltpu.async_copy(
                x_hbm.at[pl.ds((base + _CHUNK) * 128, _CW)],
                xv.at[pl.ds(_CW - ibuf, _CW)],
                isem)

        @pl.when(ci >= 2)
        def _drain_out():
            pltpu.make_async_copy(
                out_hbm.at[pl.ds(0, _OW)],
                ov.at[pl.ds(0, _OW)], osem).wait()

        def pass_body(pi):
            rowf = ibuf + (iota + pi * 16) * 128
            rowo = obuf + (iota + pi * 16) * 128

            @plsc.parallel_loop(
                0, 64,
                unroll=8,
                carry=(jnp.zeros((16,), jnp.float32),
                       jnp.zeros((16,), jnp.float32)),
            )
            def kloop(k, c):
                x_lo, y_lo = c
                col = (iota + k) & 63
                a = rowf + col
                x_hi = plsc.load_gather(xv, [a])
                y_hi = plsc.load_gather(xv, [a + 64])
                s = x_hi * 64.0
                ti = s.astype(jnp.int32)
                u = ti + (ti.astype(jnp.float32) < s).astype(jnp.int32)
                plsc.addupdate_scatter(hist, [u * 16 + iota], sixteens_i)
                m = (y_hi - y_lo) / (x_hi - x_lo)
                b = y_lo - m * x_lo
                maddr = col * 16 + iota
                plsc.store_scatter(mv, [maddr], m)
                plsc.store_scatter(bv, [maddr], b)
                return (x_hi, y_hi)

            x_last, y_last = kloop
            xf = plsc.load_gather(xv, [rowf + iota])
            yf = plsc.load_gather(xv, [rowf + iota + 64])
            mw = (yf - y_last) / (xf - x_last)
            bw = y_last - mw * x_last
            msk = iota > 0
            plsc.store_scatter(mv, [iota * 17], mw, mask=msk)
            plsc.store_scatter(bv, [iota * 17], bw, mask=msk)
            x0 = plsc.load_gather(xv, [rowf + 0])
            y0 = plsc.load_gather(xv, [rowf + 64])
            m0 = (y0 - 0.0) / (x0 - (-0.01))
            mv[pl.ds(0, 16)] = m0
            bv[pl.ds(0, 16)] = 0.0 - m0 * (-0.01)
            x63 = plsc.load_gather(xv, [rowf + 63])
            y63 = plsc.load_gather(xv, [rowf + 127])
            m64 = (0.0 - y63) / (1.01 - x63)
            mv[pl.ds(64 * 16, 16)] = m64
            bv[pl.ds(64 * 16, 16)] = y63 - m64 * x63

            @plsc.parallel_loop(0, 64, unroll=8, carry=zeros_i)
            def ploop(p, cnt16):
                h = hist[pl.ds(p * 16, 16)]
                hist[pl.ds(p * 16, 16)] = zeros_i
                cnt16 = cnt16 + h
                a = cnt16 + iota
                mg = plsc.load_gather(mv, [a])
                bg = plsc.load_gather(bv, [a])
                t = p.astype(jnp.float32) * (1.0 / 64.0)
                o = mg * t + bg
                ocm[pl.ds(p * 16, 16)] = o
                return cnt16

            del ploop
            hist[pl.ds(64 * 16, 16)] = zeros_i

            @plsc.parallel_loop(0, 16, unroll=4)
            def tloop(d):
                e = (iota + d) & 15
                e16 = e * 16
                for blk in range(4):
                    src = e16 + iota + blk * 256
                    v = plsc.load_gather(ocm, [src])
                    dst = rowo + e + blk * 16
                    plsc.store_scatter(ov, [dst], v)

            del tloop

        pl.loop(0, _NPASS)(pass_body)
        pltpu.async_copy(
            ov.at[pl.ds(obuf, _OW)],
            out_hbm.at[pl.ds(base * 128, _OW)],
            osem)

    pl.loop(0, _NCHUNK)(chunk_body)
    for _ in range(2):
        pltpu.make_async_copy(
            out_hbm.at[pl.ds(0, _OW)],
            ov.at[pl.ds(0, _OW)], osem).wait()


def kernel(X):
    mesh = plsc.VectorSubcoreMesh(core_axis_name="c", subcore_axis_name="s")
    f = functools.partial(
        pl.kernel,
        mesh=mesh,
        compiler_params=pltpu.CompilerParams(needs_layout_passes=False),
        out_type=jax.ShapeDtypeStruct((_B * 128,), jnp.float32),
        scratch_types=[
            pltpu.VMEM((2 * _CW,), jnp.float32),
            pltpu.VMEM((2 * _OW,), jnp.float32),
            pltpu.VMEM((64 * 16,), jnp.float32),
            pltpu.VMEM((65 * 16,), jnp.int32),
            pltpu.VMEM((65 * 16,), jnp.float32),
            pltpu.VMEM((65 * 16,), jnp.float32),
            pltpu.SemaphoreType.DMA,
            pltpu.SemaphoreType.DMA,
        ],
    )(_sc_body)
    out = f(X.reshape(_B * 128))
    return out.reshape(_B, 128)[:, :64]
